# baseline (speedup 1.0000x reference)
import functools

import jax
import jax.numpy as jnp
from jax import lax
from jax.experimental import pallas as pl
from jax.experimental.pallas import tpu as pltpu
from jax.experimental.pallas import tpu_sc as plsc

_V = 100000
_D = 300
_DP = 384
_B = 16384
_NC = 2
_NS = 16
_NW = _NC * _NS
_BPW = _B // _NW
_CHUNK = 128
_NCHUNK = _BPW // _CHUNK
_NG = (_V + 127) // 128

_mesh = plsc.VectorSubcoreMesh(core_axis_name="c", subcore_axis_name="s")


_TCB = 1024
_NGB = (_V + _TCB - 1) // _TCB


def _tc_body(in_ref, out_ref):
    blk = in_ref[...]
    rows = jax.lax.broadcasted_iota(jnp.int32, (_D, _D), 0)
    cols = jax.lax.broadcasted_iota(jnp.int32, (_D, _D), 1)
    eye = jnp.where(rows == cols, 1.0, 0.0).astype(jnp.float32)

    hi = blk.astype(jnp.bfloat16).astype(jnp.float32)
    r1 = blk - hi
    mid = r1.astype(jnp.bfloat16).astype(jnp.float32)
    lo = r1 - mid

    def tsel(x):
        return jax.lax.dot_general(
            x, eye, (((0,), (0,)), ((), ())),
            preferred_element_type=jnp.float32,
            precision=jax.lax.Precision.DEFAULT)

    out_ref[:, :_D] = (tsel(hi) + tsel(mid)) + tsel(lo)


_tc_transpose = pl.pallas_call(
    _tc_body,
    grid=(_NGB,),
    in_specs=[pl.BlockSpec((_D, _TCB), lambda c: (0, c))],
    out_specs=pl.BlockSpec((_TCB, _DP), lambda c: (c, 0)),
    out_shape=jax.ShapeDtypeStruct((_V, _DP), jnp.float32),
)


@functools.partial(
    pl.kernel,
    mesh=_mesh,
    out_type=jax.ShapeDtypeStruct((_B, _DP), jnp.float32),
    scratch_types=[
        pltpu.VMEM((_NCHUNK, _CHUNK), jnp.int32),
        pltpu.VMEM((_CHUNK, _DP), jnp.float32),
        pltpu.VMEM((_CHUNK, _DP), jnp.float32),
        pltpu.SemaphoreType.DMA,
        pltpu.SemaphoreType.DMA,
        pltpu.SemaphoreType.DMA,
        pltpu.SemaphoreType.DMA,
    ],
)
def _gather_kernel(t2_hbm, idx_hbm, out_hbm, idx_v, rows0, rows1,
                   gsem0, gsem1, osem0, osem1):
    wid = lax.axis_index("s") * _NC + lax.axis_index("c")
    base = wid * _BPW

    pltpu.sync_copy(idx_hbm.at[wid], idx_v)

    bufs = (rows0, rows1)
    gsems = (gsem0, gsem1)
    osems = (osem0, osem1)

    def start_gather(c):
        cps = []
        for t in range(3):
            cps.append(pltpu.async_copy(
                t2_hbm.at[idx_v.at[c], pl.ds(t * 128, 128)],
                bufs[c % 2].at[:, pl.ds(t * 128, 128)], gsems[c % 2]))
        return cps

    gathers = [None] * _NCHUNK
    outs = [None] * _NCHUNK
    gathers[0] = start_gather(0)
    for c in range(_NCHUNK):
        nxt = c + 1
        if nxt < _NCHUNK:
            if nxt >= 2:
                outs[nxt - 2].wait()
                outs[nxt - 2] = None
            gathers[nxt] = start_gather(nxt)
        for cp in gathers[c]:
            cp.wait()
        outs[c] = pltpu.async_copy(
            bufs[c % 2], out_hbm.at[pl.ds(base + c * _CHUNK, _CHUNK)],
            osems[c % 2])
    for c in range(_NCHUNK):
        if outs[c] is not None:
            outs[c].wait()


def kernel(table, indices):
    tt = table.T
    idx = indices.astype(jnp.int32).reshape(_NW, _NCHUNK, _CHUNK)
    t2 = _tc_transpose(tt)
    out = _gather_kernel(t2, idx)
    return out[:, :_D]


# speedup vs baseline: 1.1233x; 1.1233x over previous
import functools

import jax
import jax.numpy as jnp
from jax import lax
from jax.experimental import pallas as pl
from jax.experimental.pallas import tpu as pltpu
from jax.experimental.pallas import tpu_sc as plsc

_V = 100000
_D = 300
_DP = 384
_B = 16384
_NC = 2
_NS = 16
_NW = _NC * _NS
_BPW = _B // _NW
_CHUNK = 128
_NCHUNK = _BPW // _CHUNK
_NG = (_V + 127) // 128

_mesh = plsc.VectorSubcoreMesh(core_axis_name="c", subcore_axis_name="s")


_TCB = 2048
_NGB = (_V + _TCB - 1) // _TCB


def _tc_body(in_ref, out_ref):
    blk = in_ref[...]
    rows = jax.lax.broadcasted_iota(jnp.int32, (_D, _D), 0)
    cols = jax.lax.broadcasted_iota(jnp.int32, (_D, _D), 1)
    eye = jnp.where(rows == cols, 1.0, 0.0).astype(jnp.float32)

    hi = blk.astype(jnp.bfloat16).astype(jnp.float32)
    r1 = blk - hi
    mid = r1.astype(jnp.bfloat16).astype(jnp.float32)
    lo = r1 - mid

    def tsel(x):
        return jax.lax.dot_general(
            x, eye, (((0,), (0,)), ((), ())),
            preferred_element_type=jnp.float32,
            precision=jax.lax.Precision.DEFAULT)

    out_ref[:, :_D] = (tsel(hi) + tsel(mid)) + tsel(lo)


_tc_transpose = pl.pallas_call(
    _tc_body,
    grid=(_NGB,),
    in_specs=[pl.BlockSpec((_D, _TCB), lambda c: (0, c))],
    out_specs=pl.BlockSpec((_TCB, _DP), lambda c: (c, 0)),
    out_shape=jax.ShapeDtypeStruct((_V, _DP), jnp.float32),
)


@functools.partial(
    pl.kernel,
    mesh=_mesh,
    out_type=jax.ShapeDtypeStruct((_B, _DP), jnp.float32),
    scratch_types=[
        pltpu.VMEM((_NCHUNK, _CHUNK), jnp.int32),
        pltpu.VMEM((_CHUNK, _DP), jnp.float32),
        pltpu.VMEM((_CHUNK, _DP), jnp.float32),
        pltpu.SemaphoreType.DMA,
        pltpu.SemaphoreType.DMA,
        pltpu.SemaphoreType.DMA,
        pltpu.SemaphoreType.DMA,
    ],
)
def _gather_kernel(t2_hbm, idx_hbm, out_hbm, idx_v, rows0, rows1,
                   gsem0, gsem1, osem0, osem1):
    wid = lax.axis_index("s") * _NC + lax.axis_index("c")
    base = wid * _BPW

    pltpu.sync_copy(idx_hbm.at[wid], idx_v)

    bufs = (rows0, rows1)
    gsems = (gsem0, gsem1)
    osems = (osem0, osem1)

    def start_gather(c):
        cps = []
        for t in range(3):
            cps.append(pltpu.async_copy(
                t2_hbm.at[idx_v.at[c], pl.ds(t * 128, 128)],
                bufs[c % 2].at[:, pl.ds(t * 128, 128)], gsems[c % 2]))
        return cps

    gathers = [None] * _NCHUNK
    outs = [None] * _NCHUNK
    gathers[0] = start_gather(0)
    for c in range(_NCHUNK):
        nxt = c + 1
        if nxt < _NCHUNK:
            if nxt >= 2:
                outs[nxt - 2].wait()
                outs[nxt - 2] = None
            gathers[nxt] = start_gather(nxt)
        for cp in gathers[c]:
            cp.wait()
        outs[c] = pltpu.async_copy(
            bufs[c % 2], out_hbm.at[pl.ds(base + c * _CHUNK, _CHUNK)],
            osems[c % 2])
    for c in range(_NCHUNK):
        if outs[c] is not None:
            outs[c].wait()


def kernel(table, indices):
    tt = table.T
    idx = indices.astype(jnp.int32).reshape(_NW, _NCHUNK, _CHUNK)
    t2 = _tc_transpose(tt)
    out = _gather_kernel(t2, idx)
    return out[:, :_D]


# speedup vs baseline: 1.1402x; 1.0151x over previous
import functools

import jax
import jax.numpy as jnp
from jax import lax
from jax.experimental import pallas as pl
from jax.experimental.pallas import tpu as pltpu
from jax.experimental.pallas import tpu_sc as plsc

_V = 100000
_D = 300
_DP = 384
_B = 16384
_NC = 2
_NS = 16
_NW = _NC * _NS
_BPW = _B // _NW
_CHUNK = 128
_NCHUNK = _BPW // _CHUNK
_NG = (_V + 127) // 128

_mesh = plsc.VectorSubcoreMesh(core_axis_name="c", subcore_axis_name="s")


_TCB = 4096
_NGB = (_V + _TCB - 1) // _TCB


def _tc_body(in_ref, out_ref):
    blk = in_ref[...]
    rows = jax.lax.broadcasted_iota(jnp.int32, (_D, _D), 0)
    cols = jax.lax.broadcasted_iota(jnp.int32, (_D, _D), 1)
    eye = jnp.where(rows == cols, 1.0, 0.0).astype(jnp.float32)

    hi = blk.astype(jnp.bfloat16).astype(jnp.float32)
    r1 = blk - hi
    mid = r1.astype(jnp.bfloat16).astype(jnp.float32)
    lo = r1 - mid

    def tsel(x):
        return jax.lax.dot_general(
            x, eye, (((0,), (0,)), ((), ())),
            preferred_element_type=jnp.float32,
            precision=jax.lax.Precision.DEFAULT)

    out_ref[:, :_D] = (tsel(hi) + tsel(mid)) + tsel(lo)


_tc_transpose = pl.pallas_call(
    _tc_body,
    grid=(_NGB,),
    in_specs=[pl.BlockSpec((_D, _TCB), lambda c: (0, c))],
    out_specs=pl.BlockSpec((_TCB, _DP), lambda c: (c, 0)),
    out_shape=jax.ShapeDtypeStruct((_V, _DP), jnp.float32),
)


@functools.partial(
    pl.kernel,
    mesh=_mesh,
    out_type=jax.ShapeDtypeStruct((_B, _DP), jnp.float32),
    scratch_types=[
        pltpu.VMEM((_NCHUNK, _CHUNK), jnp.int32),
        pltpu.VMEM((_CHUNK, _DP), jnp.float32),
        pltpu.VMEM((_CHUNK, _DP), jnp.float32),
        pltpu.SemaphoreType.DMA,
        pltpu.SemaphoreType.DMA,
        pltpu.SemaphoreType.DMA,
        pltpu.SemaphoreType.DMA,
    ],
)
def _gather_kernel(t2_hbm, idx_hbm, out_hbm, idx_v, rows0, rows1,
                   gsem0, gsem1, osem0, osem1):
    wid = lax.axis_index("s") * _NC + lax.axis_index("c")
    base = wid * _BPW

    pltpu.sync_copy(idx_hbm.at[wid], idx_v)

    bufs = (rows0, rows1)
    gsems = (gsem0, gsem1)
    osems = (osem0, osem1)

    def start_gather(c):
        cps = []
        for t in range(3):
            cps.append(pltpu.async_copy(
                t2_hbm.at[idx_v.at[c], pl.ds(t * 128, 128)],
                bufs[c % 2].at[:, pl.ds(t * 128, 128)], gsems[c % 2]))
        return cps

    gathers = [None] * _NCHUNK
    outs = [None] * _NCHUNK
    gathers[0] = start_gather(0)
    for c in range(_NCHUNK):
        nxt = c + 1
        if nxt < _NCHUNK:
            if nxt >= 2:
                outs[nxt - 2].wait()
                outs[nxt - 2] = None
            gathers[nxt] = start_gather(nxt)
        for cp in gathers[c]:
            cp.wait()
        outs[c] = pltpu.async_copy(
            bufs[c % 2], out_hbm.at[pl.ds(base + c * _CHUNK, _CHUNK)],
            osems[c % 2])
    for c in range(_NCHUNK):
        if outs[c] is not None:
            outs[c].wait()


def kernel(table, indices):
    tt = table.T
    idx = indices.astype(jnp.int32).reshape(_NW, _NCHUNK, _CHUNK)
    t2 = _tc_transpose(tt)
    out = _gather_kernel(t2, idx)
    return out[:, :_D]


# speedup vs baseline: 1.2968x; 1.1373x over previous
import functools

import jax
import jax.numpy as jnp
from jax import lax
from jax.experimental import pallas as pl
from jax.experimental.pallas import tpu as pltpu
from jax.experimental.pallas import tpu_sc as plsc

_V = 100000
_D = 300
_DP = 384
_B = 16384
_NC = 2
_NS = 16
_NW = _NC * _NS
_BPW = _B // _NW
_CHUNK = 128
_NCHUNK = _BPW // _CHUNK
_NG = (_V + 127) // 128

_mesh = plsc.VectorSubcoreMesh(core_axis_name="c", subcore_axis_name="s")


_TCB = 4096
_NGB = (_V + _TCB - 1) // _TCB


def _tc_body(in_ref, out_ref):
    blk = in_ref[...]
    rows = jax.lax.broadcasted_iota(jnp.int32, (3 * _D, _D), 0)
    cols = jax.lax.broadcasted_iota(jnp.int32, (3 * _D, _D), 1)
    eye3 = jnp.where(rows % _D == cols, 1.0, 0.0).astype(jnp.float32)

    hi = blk.astype(jnp.bfloat16).astype(jnp.float32)
    r1 = blk - hi
    mid = r1.astype(jnp.bfloat16).astype(jnp.float32)
    lo = r1 - mid
    stacked = jnp.concatenate([hi, mid, lo], axis=0)

    out_ref[:, :_D] = jax.lax.dot_general(
        stacked, eye3, (((0,), (0,)), ((), ())),
        preferred_element_type=jnp.float32,
        precision=jax.lax.Precision.DEFAULT)


_tc_transpose = pl.pallas_call(
    _tc_body,
    grid=(_NGB,),
    in_specs=[pl.BlockSpec((_D, _TCB), lambda c: (0, c))],
    out_specs=pl.BlockSpec((_TCB, _DP), lambda c: (c, 0)),
    out_shape=jax.ShapeDtypeStruct((_V, _DP), jnp.float32),
)


@functools.partial(
    pl.kernel,
    mesh=_mesh,
    out_type=jax.ShapeDtypeStruct((_B, _DP), jnp.float32),
    scratch_types=[
        pltpu.VMEM((_NCHUNK, _CHUNK), jnp.int32),
        pltpu.VMEM((_CHUNK, _DP), jnp.float32),
        pltpu.VMEM((_CHUNK, _DP), jnp.float32),
        pltpu.SemaphoreType.DMA,
        pltpu.SemaphoreType.DMA,
        pltpu.SemaphoreType.DMA,
        pltpu.SemaphoreType.DMA,
    ],
)
def _gather_kernel(t2_hbm, idx_hbm, out_hbm, idx_v, rows0, rows1,
                   gsem0, gsem1, osem0, osem1):
    wid = lax.axis_index("s") * _NC + lax.axis_index("c")
    base = wid * _BPW

    pltpu.sync_copy(idx_hbm.at[wid], idx_v)

    bufs = (rows0, rows1)
    gsems = (gsem0, gsem1)
    osems = (osem0, osem1)

    def start_gather(c):
        cps = []
        for t in range(3):
            cps.append(pltpu.async_copy(
                t2_hbm.at[idx_v.at[c], pl.ds(t * 128, 128)],
                bufs[c % 2].at[:, pl.ds(t * 128, 128)], gsems[c % 2]))
        return cps

    gathers = [None] * _NCHUNK
    outs = [None] * _NCHUNK
    gathers[0] = start_gather(0)
    for c in range(_NCHUNK):
        nxt = c + 1
        if nxt < _NCHUNK:
            if nxt >= 2:
                outs[nxt - 2].wait()
                outs[nxt - 2] = None
            gathers[nxt] = start_gather(nxt)
        for cp in gathers[c]:
            cp.wait()
        outs[c] = pltpu.async_copy(
            bufs[c % 2], out_hbm.at[pl.ds(base + c * _CHUNK, _CHUNK)],
            osems[c % 2])
    for c in range(_NCHUNK):
        if outs[c] is not None:
            outs[c].wait()


def kernel(table, indices):
    tt = table.T
    idx = indices.astype(jnp.int32).reshape(_NW, _NCHUNK, _CHUNK)
    t2 = _tc_transpose(tt)
    out = _gather_kernel(t2, idx)
    return out[:, :_D]


# speedup vs baseline: 1.2981x; 1.0010x over previous
import functools

import jax
import jax.numpy as jnp
from jax import lax
from jax.experimental import pallas as pl
from jax.experimental.pallas import tpu as pltpu
from jax.experimental.pallas import tpu_sc as plsc

_V = 100000
_D = 300
_DP = 384
_B = 16384
_NC = 2
_NS = 16
_NW = _NC * _NS
_BPW = _B // _NW
_CHUNK = 128
_NCHUNK = _BPW // _CHUNK

_mesh = plsc.VectorSubcoreMesh(core_axis_name="c", subcore_axis_name="s")

_TCB = 4096
_NGB = (_V + _TCB - 1) // _TCB


def _tc_body(in_ref, out_ref):
    blk = in_ref[...]
    rows = jax.lax.broadcasted_iota(jnp.int32, (3 * _D, _D), 0)
    cols = jax.lax.broadcasted_iota(jnp.int32, (3 * _D, _D), 1)
    eye3 = jnp.where(rows % _D == cols, 1.0, 0.0).astype(jnp.float32)

    hi = blk.astype(jnp.bfloat16).astype(jnp.float32)
    r1 = blk - hi
    mid = r1.astype(jnp.bfloat16).astype(jnp.float32)
    lo = r1 - mid
    stacked = jnp.concatenate([hi, mid, lo], axis=0)

    out_ref[:, :_D] = jax.lax.dot_general(
        stacked, eye3, (((0,), (0,)), ((), ())),
        preferred_element_type=jnp.float32,
        precision=jax.lax.Precision.DEFAULT)


_tc_transpose = pl.pallas_call(
    _tc_body,
    grid=(_NGB,),
    in_specs=[pl.BlockSpec((_D, _TCB), lambda c: (0, c))],
    out_specs=pl.BlockSpec((_TCB, _DP), lambda c: (c, 0)),
    out_shape=jax.ShapeDtypeStruct((_V, _DP), jnp.float32),
)


@functools.partial(
    pl.kernel,
    mesh=_mesh,
    out_type=jax.ShapeDtypeStruct((_B, _DP), jnp.float32),
    scratch_types=[
        pltpu.VMEM((_NCHUNK, _CHUNK), jnp.int32),
        pltpu.VMEM((_CHUNK, _DP), jnp.float32),
        pltpu.VMEM((_CHUNK, _DP), jnp.float32),
        pltpu.SemaphoreType.DMA,
        pltpu.SemaphoreType.DMA,
        pltpu.SemaphoreType.DMA,
        pltpu.SemaphoreType.DMA,
    ],
)
def _gather_kernel(t2_hbm, idx_hbm, out_hbm, idx_v, rows0, rows1,
                   gsem0, gsem1, osem0, osem1):
    wid = lax.axis_index("s") * _NC + lax.axis_index("c")
    base = wid * _BPW

    pltpu.sync_copy(idx_hbm.at[wid], idx_v)

    bufs = (rows0, rows1)
    gsems = (gsem0, gsem1)
    osems = (osem0, osem1)

    def start_gather(c):
        cps = []
        for t in range(3):
            cps.append(pltpu.async_copy(
                t2_hbm.at[idx_v.at[c], pl.ds(t * 128, 128)],
                bufs[c % 2].at[:, pl.ds(t * 128, 128)], gsems[c % 2]))
        return cps

    gathers = [None] * _NCHUNK
    outs = [None] * _NCHUNK
    gathers[0] = start_gather(0)
    for c in range(_NCHUNK):
        nxt = c + 1
        if nxt < _NCHUNK:
            if nxt >= 2:
                outs[nxt - 2].wait()
                outs[nxt - 2] = None
            gathers[nxt] = start_gather(nxt)
        for cp in gathers[c]:
            cp.wait()
        outs[c] = pltpu.async_copy(
            bufs[c % 2], out_hbm.at[pl.ds(base + c * _CHUNK, _CHUNK)],
            osems[c % 2])
    for c in range(_NCHUNK):
        if outs[c] is not None:
            outs[c].wait()


def kernel(table, indices):
    tt = table.T
    idx = indices.astype(jnp.int32).reshape(_NW, _NCHUNK, _CHUNK)
    t2 = _tc_transpose(tt)
    out = _gather_kernel(t2, idx)
    return out[:, :_D]


# speedup vs baseline: 1.2999x; 1.0014x over previous
import functools

import jax
import jax.numpy as jnp
from jax import lax
from jax.experimental import pallas as pl
from jax.experimental.pallas import tpu as pltpu
from jax.experimental.pallas import tpu_sc as plsc

_V = 100000
_D = 300
_DP = 384
_B = 16384
_NC = 2
_NS = 16
_NW = _NC * _NS
_BPW = _B // _NW
_CHUNK = 128
_NCHUNK = _BPW // _CHUNK

_mesh = plsc.VectorSubcoreMesh(core_axis_name="c", subcore_axis_name="s")

_TCB = 4096
_NGB = (_V + _TCB - 1) // _TCB


def _tc_body(in_ref, out_ref):
    blk = in_ref[...]
    rows = jax.lax.broadcasted_iota(jnp.int32, (3 * _D, _D), 0)
    cols = jax.lax.broadcasted_iota(jnp.int32, (3 * _D, _D), 1)
    eye3 = jnp.where(rows % _D == cols, 1.0, 0.0).astype(jnp.float32)

    hi = blk.astype(jnp.bfloat16).astype(jnp.float32)
    r1 = blk - hi
    mid = r1.astype(jnp.bfloat16).astype(jnp.float32)
    lo = r1 - mid
    stacked = jnp.concatenate([hi, mid, lo], axis=0)

    out_ref[:, :_D] = jax.lax.dot_general(
        stacked, eye3, (((0,), (0,)), ((), ())),
        preferred_element_type=jnp.float32,
        precision=jax.lax.Precision.DEFAULT)


_tc_transpose = pl.pallas_call(
    _tc_body,
    grid=(_NGB,),
    in_specs=[pl.BlockSpec((_D, _TCB), lambda c: (0, c))],
    out_specs=pl.BlockSpec((_TCB, _DP), lambda c: (c, 0)),
    out_shape=jax.ShapeDtypeStruct((_V, _DP), jnp.float32),
)


@functools.partial(
    pl.kernel,
    mesh=_mesh,
    out_type=jax.ShapeDtypeStruct((_B, _DP), jnp.float32),
    scratch_types=[
        pltpu.VMEM((_NCHUNK, _CHUNK), jnp.int32),
        pltpu.VMEM((_CHUNK, _DP), jnp.float32),
        pltpu.VMEM((_CHUNK, _DP), jnp.float32),
        pltpu.SemaphoreType.DMA,
        pltpu.SemaphoreType.DMA,
        pltpu.SemaphoreType.DMA,
        pltpu.SemaphoreType.DMA,
    ],
)
def _gather_kernel(t2_hbm, idx_hbm, out_hbm, idx_v, rows0, rows1,
                   gsem0, gsem1, osem0, osem1):
    wid = lax.axis_index("s") * _NC + lax.axis_index("c")
    base = wid * _BPW

    pltpu.sync_copy(idx_hbm.at[wid], idx_v)

    bufs = (rows0, rows1)
    gsems = (gsem0, gsem1)
    osems = (osem0, osem1)

    def start_gather(c):
        return [pltpu.async_copy(
            t2_hbm.at[idx_v.at[c]], bufs[c % 2], gsems[c % 2])]

    gathers = [None] * _NCHUNK
    outs = [None] * _NCHUNK
    gathers[0] = start_gather(0)
    for c in range(_NCHUNK):
        nxt = c + 1
        if nxt < _NCHUNK:
            if nxt >= 2:
                outs[nxt - 2].wait()
                outs[nxt - 2] = None
            gathers[nxt] = start_gather(nxt)
        for cp in gathers[c]:
            cp.wait()
        outs[c] = pltpu.async_copy(
            bufs[c % 2], out_hbm.at[pl.ds(base + c * _CHUNK, _CHUNK)],
            osems[c % 2])
    for c in range(_NCHUNK):
        if outs[c] is not None:
            outs[c].wait()


def kernel(table, indices):
    tt = table.T
    idx = indices.astype(jnp.int32).reshape(_NW, _NCHUNK, _CHUNK)
    t2 = _tc_transpose(tt)
    out = _gather_kernel(t2, idx)
    return out[:, :_D]
